# baseline (speedup 1.0000x reference)
import functools

import jax
import jax.numpy as jnp
from jax import lax
from jax.experimental import pallas as pl
from jax.experimental.pallas import tpu as pltpu
from jax.experimental.pallas import tpu_sc as plsc

B = 16384
K = 8192
D = 64
DP = 128
BT = 1024
NB = B // BT
EPS = 1e-12

NC = 2
NS = 16
NW = NC * NS
BPW = B // NW
CH = 128
NCH = BPW // CH


def _prep_body(embt_ref, en_ref):
    et = embt_ref[...]
    etn = et / jnp.maximum(
        jnp.sqrt(jnp.sum(et * et, axis=0, keepdims=True)), EPS)
    en_ref[:, :D] = etn.T
    en_ref[:, D:] = jnp.zeros((K, DP - D), jnp.float32)


def _make_prep(interpret=False):
    return pl.pallas_call(
        _prep_body,
        out_shape=jax.ShapeDtypeStruct((K, DP), jnp.float32),
        interpret=interpret,
    )


def _main_body(zt_ref, en_ref, znt_ref, idx_ref):
    zt = zt_ref[...]
    zn = zt / jnp.maximum(jnp.sqrt(jnp.sum(zt * zt, axis=0, keepdims=True)),
                          EPS)
    znt_ref[...] = zn
    en = en_ref[:, :D]
    sim = lax.dot_general(zn, en, (((0,), (1,)), ((), ())),
                          preferred_element_type=jnp.float32)
    idx = jnp.argmax(sim, axis=1).astype(jnp.int32)
    idx_ref[...] = idx


def _make_main(interpret=False):
    return pl.pallas_call(
        _main_body,
        grid=(NB,),
        in_specs=[
            pl.BlockSpec((D, BT), lambda i: (0, i)),
            pl.BlockSpec((K, DP), lambda i: (0, 0)),
        ],
        out_specs=[
            pl.BlockSpec((D, BT), lambda i: (0, i)),
            pl.BlockSpec((BT,), lambda i: (i,)),
        ],
        out_shape=[
            jax.ShapeDtypeStruct((D, B), jnp.float32),
            jax.ShapeDtypeStruct((B,), jnp.int32),
        ],
        compiler_params=pltpu.CompilerParams(
            dimension_semantics=("parallel",)),
        interpret=interpret,
    )


def _gat_body(tab_ref, idx_ref, zq_ref, idxv, rows, sem):
    cid = lax.axis_index("c")
    sid = lax.axis_index("s")
    wid = sid * NC + cid
    pltpu.sync_copy(idx_ref.at[pl.ds(wid * NCH, NCH)], idxv)
    for j in range(NCH):
        pltpu.async_copy(tab_ref.at[idxv.at[j]],
                         rows.at[pl.ds(j * CH, CH)], sem).wait()
    pltpu.sync_copy(rows, zq_ref.at[pl.ds(wid * BPW, BPW)])


def _make_gat():
    mesh = plsc.VectorSubcoreMesh(core_axis_name="c", subcore_axis_name="s")
    return pl.kernel(
        _gat_body,
        mesh=mesh,
        out_type=jax.ShapeDtypeStruct((B, DP), jnp.float32),
        scratch_types=[
            pltpu.VMEM((NCH, CH), jnp.int32),
            pltpu.VMEM((BPW, DP), jnp.float32),
            pltpu.SemaphoreType.DMA,
        ],
    )


def _hist_body(idx_ref, cnt_ref, idxv, ones_v, zer_v, cnt_sh):
    cid = lax.axis_index("c")
    sid = lax.axis_index("s")
    wid = sid * NC + cid
    pltpu.sync_copy(idx_ref.at[pl.ds(wid * NCH, NCH)], idxv)
    for j in range(CH // 16):
        ones_v[pl.ds(j * 16, 16)] = jnp.ones((16,), jnp.float32)
    for j in range((K // NS) // 16):
        zer_v[pl.ds(j * 16, 16)] = jnp.zeros((16,), jnp.float32)
    pltpu.sync_copy(zer_v, cnt_sh.at[pl.ds(sid * (K // NS), K // NS)])
    plsc.subcore_barrier()
    for j in range(NCH):
        pltpu.sync_copy(ones_v, cnt_sh.at[idxv.at[j]], add=True)
    plsc.subcore_barrier()

    @pl.when(sid == 0)
    def _():
        pltpu.sync_copy(cnt_sh, cnt_ref.at[cid])


def _make_hist():
    mesh = plsc.VectorSubcoreMesh(core_axis_name="c", subcore_axis_name="s")
    return pl.kernel(
        _hist_body,
        mesh=mesh,
        out_type=jax.ShapeDtypeStruct((NC, K), jnp.float32),
        scratch_types=[
            pltpu.VMEM((NCH, CH), jnp.int32),
            pltpu.VMEM((CH,), jnp.float32),
            pltpu.VMEM((K // NS,), jnp.float32),
            pltpu.VMEM_SHARED((K,), jnp.float32),
        ],
        compiler_params=pltpu.CompilerParams(use_tc_tiling_on_sc=False),
    )


FT = 2048
NF = B // FT


def _fin_body(zn_ref, zq_ref, zste_ref, com_ref, cod_ref, acc_ref):
    i = pl.program_id(0)
    zn = zn_ref[...]
    zq = zq_ref[:, :D].T
    zste_ref[...] = zn + (zq - zn)
    dlt = zn - zq
    s = jnp.sum(dlt * dlt)

    @pl.when(i == 0)
    def _():
        acc_ref[0] = s

    @pl.when(i > 0)
    def _():
        acc_ref[0] += s

    @pl.when(i == NF - 1)
    def _():
        mse = acc_ref[0] * (1.0 / (B * D))
        com_ref[0, 0] = 0.25 * mse
        cod_ref[0, 0] = mse


def _make_fin(interpret=False):
    scalar = jax.ShapeDtypeStruct((1, 1), jnp.float32)
    smem = pl.BlockSpec((1, 1), lambda i: (0, 0), memory_space=pltpu.SMEM)
    return pl.pallas_call(
        _fin_body,
        grid=(NF,),
        in_specs=[
            pl.BlockSpec((D, FT), lambda i: (0, i)),
            pl.BlockSpec((FT, DP), lambda i: (i, 0)),
        ],
        out_specs=[pl.BlockSpec((D, FT), lambda i: (0, i)),
                   smem, smem],
        out_shape=[jax.ShapeDtypeStruct((D, B), jnp.float32),
                   scalar, scalar],
        scratch_shapes=[pltpu.SMEM((1,), jnp.float32)],
        interpret=interpret,
    )


def _pp_body(cnt_ref, per_ref, util_ref):
    c = cnt_ref[...]
    cc = c[0:1, :] + c[1:2, :]
    p = cc * (1.0 / B)
    ent = -jnp.sum(p * jnp.log(p + 1e-10))
    per_ref[0, 0] = jnp.exp(ent)
    util_ref[0, 0] = jnp.sum((cc > 0.0).astype(jnp.float32)) * (1.0 / K)


def _make_pp(interpret=False):
    scalar = jax.ShapeDtypeStruct((1, 1), jnp.float32)
    smem = pl.BlockSpec(memory_space=pltpu.SMEM)
    return pl.pallas_call(
        _pp_body,
        out_specs=[smem, smem],
        out_shape=[scalar, scalar],
        interpret=interpret,
    )


_prep = _make_prep()
_main = _make_main()
_fin = _make_fin()
_pp = _make_pp()
_get_gat = functools.cache(_make_gat)
_get_hist = functools.cache(_make_hist)


def kernel(z_e, embeddings):
    en128 = _prep(embeddings.T)
    znt, idx = _main(z_e.T, en128)
    idx2 = idx.reshape(B // CH, CH)
    zq = _get_gat()(en128, idx2)
    cnt2 = _get_hist()(idx2)
    zstet, com, cod = _fin(znt, zq)
    per, util = _pp(cnt2)
    return (zstet.T, idx, com.reshape(()), cod.reshape(()),
            per.reshape(()), util.reshape(()))


# speedup vs baseline: 1.0150x; 1.0150x over previous
import functools

import jax
import jax.numpy as jnp
from jax import lax
from jax.experimental import pallas as pl
from jax.experimental.pallas import tpu as pltpu
from jax.experimental.pallas import tpu_sc as plsc

B = 16384
K = 8192
D = 64
DP = 128
BT = 1024
NB = B // BT
EPS = 1e-12

NC = 2
NS = 16
NW = NC * NS
BPW = B // NW
CH = 128
NCH = BPW // CH


def _prep_body(embt_ref, en_ref):
    et = embt_ref[...]
    etn = et / jnp.maximum(
        jnp.sqrt(jnp.sum(et * et, axis=0, keepdims=True)), EPS)
    en_ref[:, :D] = etn.T
    en_ref[:, D:] = jnp.zeros((K, DP - D), jnp.float32)


def _make_prep(interpret=False):
    return pl.pallas_call(
        _prep_body,
        out_shape=jax.ShapeDtypeStruct((K, DP), jnp.float32),
        interpret=interpret,
    )


def _main_body(zt_ref, en_ref, znt_ref, idx_ref):
    zt = zt_ref[...]
    zn = zt / jnp.maximum(jnp.sqrt(jnp.sum(zt * zt, axis=0, keepdims=True)),
                          EPS)
    znt_ref[...] = zn
    en = en_ref[:, :D]
    sim = lax.dot_general(zn, en, (((0,), (1,)), ((), ())),
                          preferred_element_type=jnp.float32)
    idx = jnp.argmax(sim, axis=1).astype(jnp.int32)
    idx_ref[...] = idx


def _make_main(interpret=False):
    return pl.pallas_call(
        _main_body,
        grid=(NB,),
        in_specs=[
            pl.BlockSpec((D, BT), lambda i: (0, i)),
            pl.BlockSpec((K, DP), lambda i: (0, 0)),
        ],
        out_specs=[
            pl.BlockSpec((D, BT), lambda i: (0, i)),
            pl.BlockSpec((BT,), lambda i: (i,)),
        ],
        out_shape=[
            jax.ShapeDtypeStruct((D, B), jnp.float32),
            jax.ShapeDtypeStruct((B,), jnp.int32),
        ],
        compiler_params=pltpu.CompilerParams(
            dimension_semantics=("parallel",)),
        interpret=interpret,
    )


def _gat_body(tab_ref, idx_ref, zq_ref, idxv, rows, sem):
    cid = lax.axis_index("c")
    sid = lax.axis_index("s")
    wid = sid * NC + cid
    pltpu.sync_copy(idx_ref.at[pl.ds(wid * NCH, NCH)], idxv)
    for j in range(NCH):
        pltpu.async_copy(tab_ref.at[idxv.at[j]],
                         rows.at[pl.ds(j * CH, CH)], sem).wait()
    pltpu.sync_copy(rows, zq_ref.at[pl.ds(wid * BPW, BPW)])


def _make_gat():
    mesh = plsc.VectorSubcoreMesh(core_axis_name="c", subcore_axis_name="s")
    return pl.kernel(
        _gat_body,
        mesh=mesh,
        out_type=jax.ShapeDtypeStruct((B, DP), jnp.float32),
        scratch_types=[
            pltpu.VMEM((NCH, CH), jnp.int32),
            pltpu.VMEM((BPW, DP), jnp.float32),
            pltpu.SemaphoreType.DMA,
        ],
    )


def _hist_body(idx_ref, cnt_ref, idxv, ones_v, zer_v, cnt_sh):
    cid = lax.axis_index("c")
    sid = lax.axis_index("s")
    wid = sid * NC + cid
    pltpu.sync_copy(idx_ref.at[pl.ds(wid * NCH, NCH)], idxv)
    for j in range(CH // 16):
        ones_v[pl.ds(j * 16, 16)] = jnp.ones((16,), jnp.float32)
    for j in range((K // NS) // 16):
        zer_v[pl.ds(j * 16, 16)] = jnp.zeros((16,), jnp.float32)
    pltpu.sync_copy(zer_v, cnt_sh.at[pl.ds(sid * (K // NS), K // NS)])
    plsc.subcore_barrier()
    for j in range(NCH):
        pltpu.sync_copy(ones_v, cnt_sh.at[idxv.at[j]], add=True)
    plsc.subcore_barrier()

    @pl.when(sid == 0)
    def _():
        pltpu.sync_copy(cnt_sh, cnt_ref.at[cid])


def _make_hist():
    mesh = plsc.VectorSubcoreMesh(core_axis_name="c", subcore_axis_name="s")
    return pl.kernel(
        _hist_body,
        mesh=mesh,
        out_type=jax.ShapeDtypeStruct((NC, K), jnp.float32),
        scratch_types=[
            pltpu.VMEM((NCH, CH), jnp.int32),
            pltpu.VMEM((CH,), jnp.float32),
            pltpu.VMEM((K // NS,), jnp.float32),
            pltpu.VMEM_SHARED((K,), jnp.float32),
        ],
        compiler_params=pltpu.CompilerParams(use_tc_tiling_on_sc=False),
    )


FT = 2048
NF = B // FT


def _fin_body(zn_ref, zq_ref, zste_ref, com_ref, cod_ref, acc_ref):
    i = pl.program_id(0)
    zn = zn_ref[...]
    zq = zq_ref[:, :D].T
    zste_ref[...] = zn + (zq - zn)
    dlt = zn - zq
    s = jnp.sum(dlt * dlt)

    @pl.when(i == 0)
    def _():
        acc_ref[0] = s

    @pl.when(i > 0)
    def _():
        acc_ref[0] += s

    @pl.when(i == NF - 1)
    def _():
        mse = acc_ref[0] * (1.0 / (B * D))
        com_ref[0, 0] = 0.25 * mse
        cod_ref[0, 0] = mse


def _make_fin(interpret=False):
    scalar = jax.ShapeDtypeStruct((1, 1), jnp.float32)
    smem = pl.BlockSpec((1, 1), lambda i: (0, 0), memory_space=pltpu.SMEM)
    return pl.pallas_call(
        _fin_body,
        grid=(NF,),
        in_specs=[
            pl.BlockSpec((D, FT), lambda i: (0, i)),
            pl.BlockSpec((FT, DP), lambda i: (i, 0)),
        ],
        out_specs=[pl.BlockSpec((D, FT), lambda i: (0, i)),
                   smem, smem],
        out_shape=[jax.ShapeDtypeStruct((D, B), jnp.float32),
                   scalar, scalar],
        scratch_shapes=[pltpu.SMEM((1,), jnp.float32)],
        interpret=interpret,
    )


def _pp_body(cnt_ref, per_ref, util_ref):
    c = cnt_ref[...]
    cc = c[0:1, :] + c[1:2, :]
    p = cc * (1.0 / B)
    ent = -jnp.sum(p * jnp.log(p + 1e-10))
    per_ref[0, 0] = jnp.exp(ent)
    util_ref[0, 0] = jnp.sum((cc > 0.0).astype(jnp.float32)) * (1.0 / K)


def _make_pp(interpret=False):
    scalar = jax.ShapeDtypeStruct((1, 1), jnp.float32)
    smem = pl.BlockSpec(memory_space=pltpu.SMEM)
    return pl.pallas_call(
        _pp_body,
        out_specs=[smem, smem],
        out_shape=[scalar, scalar],
        interpret=interpret,
    )


_prep = _make_prep()
_main = _make_main()
_fin = _make_fin()
_pp = _make_pp()
_get_gat = functools.cache(_make_gat)
_get_hist = functools.cache(_make_hist)


def kernel(z_e, embeddings):
    en128 = _prep(embeddings.T)
    znt, idx = _main(z_e.T, en128)
    idx2 = idx.reshape(B // CH, CH)
    zq = _get_gat()(en128, idx2)
    idx2h, _ = lax.optimization_barrier((idx2, zq))
    cnt2 = _get_hist()(idx2h)
    zstet, com, cod = _fin(znt, zq)
    per, util = _pp(cnt2)
    return (zstet.T, idx, com.reshape(()), cod.reshape(()),
            per.reshape(()), util.reshape(()))
